# Initial kernel scaffold; baseline (speedup 1.0000x reference)
import jax
import jax.numpy as jnp
from jax.experimental import pallas as pl


def kernel(Z):
    raise NotImplementedError("write your pallas kernel here")


# speedup vs baseline: 5.8227x; 5.8227x over previous
import functools

import jax
import jax.numpy as jnp
import numpy as np
from jax import lax
from jax.experimental import pallas as pl
from jax.experimental.pallas import tpu as pltpu
from jax.experimental.pallas import tpu_sc as plsc

B = 8
N = 50000
D = 128
K = 2048

SH = 12504
SH_LAST = N - 3 * SH
NV = (SH + 15) // 16
NQP = NV * 16
NCH = (NQP + 127) // 128
KBUF = NCH * 128
CAND = 1024
M = 4 * CAND
BINS = 2048
INT_MIN = np.int32(-2147483648)
KEY_PAD = np.int32(-1)
IDX_PAD = np.int32(0x7FFFFFFF)

_mesh = plsc.VectorSubcoreMesh(core_axis_name="c", subcore_axis_name="s")
_params = pltpu.CompilerParams(needs_layout_passes=False)


def _iota():
    return lax.iota(jnp.int32, 16)


def _vtake(x, i):
    return lax.gather(
        x, i[:, None],
        dimension_numbers=lax.GatherDimensionNumbers(
            offset_dims=(), collapsed_slice_dims=(0,), start_index_map=(0,)),
        slice_sizes=(1,),
        mode=lax.GatherScatterMode.PROMISE_IN_BOUNDS)


def _splat(s):
    return jnp.full((16,), s, jnp.int32)


def _runs(d_s):
    iota = _iota()
    nxt = _vtake(d_s, jnp.minimum(iota + 1, 15))
    prv = _vtake(d_s, jnp.maximum(iota - 1, 0))
    is_end = (iota == 15) | (d_s != nxt)
    is_start = (iota == 0) | (d_s != prv)
    start = plsc.cummax(jnp.where(is_start, iota, 0))
    run_rank = iota - start
    return is_end, run_rank, run_rank + 1


def _sorted_digits(d):
    iota = _iota()
    sk, _ = plsc.sort_key_val(d * 16 + iota, d)
    return lax.shift_right_logical(sk, 4), sk & 15


def _radix_pass(src_k, src_i, dst_k, dst_i, bins, shift, mask):
    nv = M // 16

    def zero_body(v, _):
        bins[pl.ds(v * 16, 16)] = jnp.zeros((16,), jnp.int32)
        return 0

    lax.fori_loop(0, BINS // 16, zero_body, 0)

    def hist_body(v, _):
        ku = src_k[pl.ds(v * 16, 16)]
        d = lax.shift_right_logical(ku, shift) & mask
        d_s, _ = _sorted_digits(d)
        is_end, _, cnt = _runs(d_s)
        plsc.addupdate_scatter(bins, [d_s], cnt, mask=is_end)
        return 0

    lax.fori_loop(0, nv, hist_body, 0)

    def scan_body(v, carry):
        h = bins[pl.ds(v * 16, 16)]
        inc = plsc.cumsum(h)
        bins[pl.ds(v * 16, 16)] = inc - h + _splat(carry)
        return carry + jnp.max(inc)

    lax.fori_loop(0, BINS // 16, scan_body, jnp.int32(0))

    def perm_body(v, _):
        ku = src_k[pl.ds(v * 16, 16)]
        ix = src_i[pl.ds(v * 16, 16)]
        d = lax.shift_right_logical(ku, shift) & mask
        d_s, ol = _sorted_digits(d)
        is_end, run_rank, cnt = _runs(d_s)
        base = plsc.load_gather(bins, [d_s])
        pos = base + run_rank
        plsc.store_scatter(dst_k, [pos], _vtake(ku, ol))
        plsc.store_scatter(dst_i, [pos], _vtake(ix, ol))
        plsc.addupdate_scatter(bins, [d_s], cnt, mask=is_end)
        return 0

    lax.fori_loop(0, nv, perm_body, 0)


@functools.partial(
    pl.kernel,
    out_type=jax.ShapeDtypeStruct((B * K,), jnp.int32),
    mesh=_mesh,
    scratch_types=[
        pltpu.VMEM((KBUF,), jnp.float32),
        pltpu.VMEM((KBUF,), jnp.int32),
        pltpu.VMEM((NQP,), jnp.int32),
        pltpu.VMEM((256,), jnp.int32),
        pltpu.VMEM((4, 256), jnp.int32),
        pltpu.VMEM((CAND,), jnp.int32),
        pltpu.VMEM((CAND,), jnp.int32),
        pltpu.VMEM((M,), jnp.int32),
        pltpu.VMEM((M,), jnp.int32),
        pltpu.VMEM((M,), jnp.int32),
        pltpu.VMEM((M,), jnp.int32),
        pltpu.VMEM((BINS,), jnp.int32),
        pltpu.VMEM_SHARED((4, 4, 256), jnp.int32),
        pltpu.VMEM_SHARED((4, M), jnp.int32),
        pltpu.VMEM_SHARED((4, M), jnp.int32),
        pltpu.SemaphoreType.DMA,
    ],
    compiler_params=_params,
)
def _select_kernel(z_flat, sel_hbm, keysf, gidx, keyu, hist, hists4,
                   cand_k, cand_i, sk0, si0, sk1, si1, bins, sp_hists,
                   sp_merge_k, sp_merge_i, sem):
    c = lax.axis_index("c")
    s = lax.axis_index("s")
    wid = c * 16 + s
    b = wid // 4
    bb = s // 4
    q = s % 4
    iota = _iota()

    row0 = b * N + q * SH
    nq = jnp.where(q == 3, SH_LAST, SH)
    nclamp = _splat(nq - 1)

    def idx_body(v, _):
        i = _splat(v * 16) + iota
        r = _splat(row0) + jnp.minimum(i, nclamp)
        gidx[pl.ds(v * 16, 16)] = r * D + (D - 1)
        return 0

    lax.fori_loop(0, KBUF // 16, idx_body, 0)

    def fire_body(ch, _):
        pltpu.async_copy(z_flat.at[gidx.at[pl.ds(ch * 128, 128)]],
                         keysf.at[pl.ds(ch * 128, 128)], sem)
        return 0

    lax.fori_loop(0, NCH, fire_body, 0)
    pltpu.make_async_copy(z_flat.at[pl.ds(0, KBUF)], keysf, sem).wait()

    def hzero(v, _):
        hist[pl.ds(v * 16, 16)] = jnp.zeros((16,), jnp.int32)
        return 0

    lax.fori_loop(0, 16, hzero, 0)

    def key_body(v, _):
        kb = lax.bitcast_convert_type(keysf[pl.ds(v * 16, 16)], jnp.int32)
        m = lax.shift_right_arithmetic(kb, 31)
        ku = kb ^ (m | INT_MIN)
        valid = (_splat(v * 16) + iota) < _splat(nq)
        ku = jnp.where(valid, ku, KEY_PAD)
        keyu[pl.ds(v * 16, 16)] = ku
        d = lax.shift_right_logical(ku, 24)
        d_s, _ = _sorted_digits(d)
        is_end, _, cnt = _runs(d_s)
        plsc.addupdate_scatter(hist, [d_s], cnt, mask=is_end)
        return 0

    lax.fori_loop(0, NV, key_body, 0)

    pltpu.sync_copy(hist, sp_hists.at[bb, q])
    plsc.subcore_barrier()
    for j in range(4):
        pltpu.sync_copy(sp_hists.at[bb, j], hists4.at[j])

    def bstar_body(v, carry):
        bstar, run = carry
        h = (hists4[0, pl.ds(v * 16, 16)] + hists4[1, pl.ds(v * 16, 16)]
             + hists4[2, pl.ds(v * 16, 16)] + hists4[3, pl.ds(v * 16, 16)])
        inc = plsc.cumsum(h) + _splat(run)
        bstar = bstar + jnp.sum((inc < K).astype(jnp.int32))
        return bstar, jnp.max(inc)

    bstar, _ = lax.fori_loop(0, 16, bstar_body,
                             (jnp.int32(0), jnp.int32(0)))
    bstar_v = _splat(bstar)

    def c1_body(v, acc):
        h = (hists4[0, pl.ds(v * 16, 16)] + hists4[1, pl.ds(v * 16, 16)]
             + hists4[2, pl.ds(v * 16, 16)] + hists4[3, pl.ds(v * 16, 16)])
        binid = _splat(v * 16) + iota
        return acc + jnp.sum(jnp.where(binid < bstar_v, h, 0))

    c1 = lax.fori_loop(0, 16, c1_body, jnp.int32(0))

    def h2zero(v, _):
        hist[pl.ds(v * 16, 16)] = jnp.zeros((16,), jnp.int32)
        return 0

    lax.fori_loop(0, 16, h2zero, 0)

    def lvl2_body(v, _):
        ku = keyu[pl.ds(v * 16, 16)]
        d1 = lax.shift_right_logical(ku, 24)
        d2 = lax.shift_right_logical(ku, 16) & 0xFF
        d2m = jnp.where(d1 == bstar_v, d2, 256)
        d_s, _ = _sorted_digits(d2m)
        is_end, _, cnt = _runs(d_s)
        ok = is_end & (d_s < 256)
        plsc.addupdate_scatter(hist, [jnp.minimum(d_s, 255)], cnt, mask=ok)
        return 0

    lax.fori_loop(0, NV, lvl2_body, 0)

    plsc.subcore_barrier()
    pltpu.sync_copy(hist, sp_hists.at[bb, q])
    plsc.subcore_barrier()
    for j in range(4):
        pltpu.sync_copy(sp_hists.at[bb, j], hists4.at[j])

    k2 = K - c1

    def b2_body(v, carry):
        b2, run = carry
        h = (hists4[0, pl.ds(v * 16, 16)] + hists4[1, pl.ds(v * 16, 16)]
             + hists4[2, pl.ds(v * 16, 16)] + hists4[3, pl.ds(v * 16, 16)])
        inc = plsc.cumsum(h) + _splat(run)
        b2 = b2 + jnp.sum((inc < _splat(k2)).astype(jnp.int32))
        return b2, jnp.max(inc)

    b2star, _ = lax.fori_loop(0, 16, b2_body,
                              (jnp.int32(0), jnp.int32(0)))
    b2star_v = _splat(b2star)

    def czero(v, _):
        cand_k[pl.ds(v * 16, 16)] = jnp.full((16,), KEY_PAD, jnp.int32)
        cand_i[pl.ds(v * 16, 16)] = jnp.full((16,), IDX_PAD, jnp.int32)
        return 0

    lax.fori_loop(0, CAND // 16, czero, 0)

    def compact_body(v, off):
        ku = keyu[pl.ds(v * 16, 16)]
        d = lax.shift_right_logical(ku, 24)
        d2 = lax.shift_right_logical(ku, 16) & 0xFF
        keep = (d < bstar_v) | ((d == bstar_v) & (d2 <= b2star_v))
        node = _splat(q * SH + v * 16) + iota

        @pl.when(off <= CAND - 16)
        def _():
            plsc.store_compressed(cand_k.at[pl.ds(off, 16)], ku, mask=keep)
            plsc.store_compressed(cand_i.at[pl.ds(off, 16)], node, mask=keep)

        npop = jnp.max(plsc.all_reduce_population_count(keep))
        return off + npop

    lax.fori_loop(0, NV, compact_body, jnp.int32(0))

    pltpu.sync_copy(cand_k, sp_merge_k.at[bb, pl.ds(q * CAND, CAND)])
    pltpu.sync_copy(cand_i, sp_merge_i.at[bb, pl.ds(q * CAND, CAND)])
    plsc.subcore_barrier()

    @pl.when(q == 0)
    def _():
        pltpu.sync_copy(sp_merge_k.at[bb], sk0)
        pltpu.sync_copy(sp_merge_i.at[bb], si0)
        _radix_pass(sk0, si0, sk1, si1, bins, 0, jnp.int32(0x7FF))
        _radix_pass(sk1, si1, sk0, si0, bins, 11, jnp.int32(0x7FF))
        _radix_pass(sk0, si0, sk1, si1, bins, 22, jnp.int32(0x3FF))
        pltpu.sync_copy(si1.at[pl.ds(0, K)], sel_hbm.at[pl.ds(b * K, K)])


@functools.partial(
    pl.kernel,
    out_type=jax.ShapeDtypeStruct((B * K, D), jnp.float32),
    mesh=_mesh,
    scratch_types=[
        pltpu.VMEM((K // 4,), jnp.int32),
        pltpu.VMEM((4, 128), jnp.int32),
        pltpu.VMEM((512, D), jnp.float32),
        pltpu.SemaphoreType.DMA,
    ],
    compiler_params=_params,
)
def _gather_kernel(z_hbm, sel_hbm, out_hbm, myidx, rowidx, rows, sem):
    c = lax.axis_index("c")
    s = lax.axis_index("s")
    wid = c * 16 + s
    b = wid // 4
    kq = K // 4

    pltpu.sync_copy(sel_hbm.at[pl.ds(wid * kq, kq)], myidx)
    base_row = b * N
    for v in range(kq // 16):
        nd = myidx[pl.ds(v * 16, 16)] + _splat(base_row)
        rowidx[v // 8, pl.ds((v % 8) * 16, 16)] = nd

    for ch in range(4):
        pltpu.async_copy(z_hbm.at[rowidx.at[ch]],
                         rows.at[pl.ds(ch * 128, 128)], sem)
    pltpu.make_async_copy(z_hbm.at[pl.ds(0, 512)], rows, sem).wait()
    pltpu.sync_copy(rows, out_hbm.at[pl.ds(wid * kq, kq)])


def kernel(Z):
    sel = _select_kernel(Z.reshape(-1))
    out = _gather_kernel(Z.reshape(B * N, D), sel)
    return out.reshape(B, K, D)


# speedup vs baseline: 5.8421x; 1.0033x over previous
import functools

import jax
import jax.numpy as jnp
import numpy as np
from jax import lax
from jax.experimental import pallas as pl
from jax.experimental.pallas import tpu as pltpu
from jax.experimental.pallas import tpu_sc as plsc

B = 8
N = 50000
D = 128
K = 2048

SH = 12504
SH_LAST = N - 3 * SH
NV = (SH + 15) // 16
NQP = NV * 16
NCH = (NQP + 127) // 128
KBUF = NCH * 128
CAND = 1024
M = 4 * CAND
BINS = 2048
BUFB = 4096
INT_MIN = np.int32(-2147483648)
KEY_PAD = np.int32(-1)
IDX_PAD = np.int32(0x7FFFFFFF)

_mesh = plsc.VectorSubcoreMesh(core_axis_name="c", subcore_axis_name="s")
_params = pltpu.CompilerParams(needs_layout_passes=False)


def _iota():
    return lax.iota(jnp.int32, 16)


def _vtake(x, i):
    return lax.gather(
        x, i[:, None],
        dimension_numbers=lax.GatherDimensionNumbers(
            offset_dims=(), collapsed_slice_dims=(0,), start_index_map=(0,)),
        slice_sizes=(1,),
        mode=lax.GatherScatterMode.PROMISE_IN_BOUNDS)


def _splat(s):
    return jnp.full((16,), s, jnp.int32)


def _runs(d_s):
    iota = _iota()
    nxt = _vtake(d_s, jnp.minimum(iota + 1, 15))
    prv = _vtake(d_s, jnp.maximum(iota - 1, 0))
    is_end = (iota == 15) | (d_s != nxt)
    is_start = (iota == 0) | (d_s != prv)
    start = plsc.cummax(jnp.where(is_start, iota, 0))
    run_rank = iota - start
    return is_end, run_rank, run_rank + 1


def _sorted_digits(d):
    iota = _iota()
    sk, _ = plsc.sort_key_val(d * 16 + iota, d)
    return lax.shift_right_logical(sk, 4), sk & 15


def _radix_pass(src_k, src_i, dst_k, dst_i, bins, shift, mask):
    nv = M // 16

    def zero_body(v, _):
        bins[pl.ds(v * 16, 16)] = jnp.zeros((16,), jnp.int32)
        return 0

    lax.fori_loop(0, BINS // 16, zero_body, 0)

    def hist_body(v, _):
        for u in range(2):
            ku = src_k[pl.ds((v * 2 + u) * 16, 16)]
            d = lax.shift_right_logical(ku, shift) & mask
            d_s, _ = _sorted_digits(d)
            is_end, _, cnt = _runs(d_s)
            plsc.addupdate_scatter(bins, [d_s], cnt, mask=is_end)
        return 0

    lax.fori_loop(0, nv // 2, hist_body, 0)

    def scan_body(v, carry):
        h = bins[pl.ds(v * 16, 16)]
        inc = plsc.cumsum(h)
        bins[pl.ds(v * 16, 16)] = inc - h + _splat(carry)
        return carry + jnp.max(inc)

    lax.fori_loop(0, BINS // 16, scan_body, jnp.int32(0))

    def perm_body(v, _):
        ku = src_k[pl.ds(v * 16, 16)]
        ix = src_i[pl.ds(v * 16, 16)]
        d = lax.shift_right_logical(ku, shift) & mask
        d_s, ol = _sorted_digits(d)
        is_end, run_rank, cnt = _runs(d_s)
        base = plsc.load_gather(bins, [d_s])
        pos = base + run_rank
        plsc.store_scatter(dst_k, [pos], _vtake(ku, ol))
        plsc.store_scatter(dst_i, [pos], _vtake(ix, ol))
        plsc.addupdate_scatter(bins, [d_s], cnt, mask=is_end)
        return 0

    lax.fori_loop(0, nv, perm_body, 0)


@functools.partial(
    pl.kernel,
    out_type=jax.ShapeDtypeStruct((B * K,), jnp.int32),
    mesh=_mesh,
    scratch_types=[
        pltpu.VMEM((KBUF,), jnp.float32),
        pltpu.VMEM((KBUF,), jnp.int32),
        pltpu.VMEM((NQP,), jnp.int32),
        pltpu.VMEM((256,), jnp.int32),
        pltpu.VMEM((4, 256), jnp.int32),
        pltpu.VMEM((CAND,), jnp.int32),
        pltpu.VMEM((CAND,), jnp.int32),
        pltpu.VMEM((M,), jnp.int32),
        pltpu.VMEM((M,), jnp.int32),
        pltpu.VMEM((M,), jnp.int32),
        pltpu.VMEM((M,), jnp.int32),
        pltpu.VMEM((BINS,), jnp.int32),
        pltpu.VMEM((BUFB,), jnp.int32),
        pltpu.VMEM_SHARED((4, 4, 256), jnp.int32),
        pltpu.VMEM_SHARED((4, M), jnp.int32),
        pltpu.VMEM_SHARED((4, M), jnp.int32),
        pltpu.SemaphoreType.DMA,
    ],
    compiler_params=_params,
)
def _select_kernel(z_flat, sel_hbm, keysf, gidx, keyu, hist, hists4,
                   cand_k, cand_i, sk0, si0, sk1, si1, bins, bufb,
                   sp_hists, sp_merge_k, sp_merge_i, sem):
    c = lax.axis_index("c")
    s = lax.axis_index("s")
    wid = c * 16 + s
    b = wid // 4
    bb = s // 4
    q = s % 4
    iota = _iota()

    row0 = b * N + q * SH
    nq = jnp.where(q == 3, SH_LAST, SH)
    nclamp = _splat(nq - 1)

    def idx_body(v, _):
        i = _splat(v * 16) + iota
        r = _splat(row0) + jnp.minimum(i, nclamp)
        gidx[pl.ds(v * 16, 16)] = r * D + (D - 1)
        return 0

    lax.fori_loop(0, KBUF // 16, idx_body, 0)

    def fire_body(ch, _):
        pltpu.async_copy(z_flat.at[gidx.at[pl.ds(ch * 128, 128)]],
                         keysf.at[pl.ds(ch * 128, 128)], sem)
        return 0

    lax.fori_loop(0, NCH, fire_body, 0)
    pltpu.make_async_copy(z_flat.at[pl.ds(0, KBUF)], keysf, sem).wait()

    def hzero(v, _):
        hist[pl.ds(v * 16, 16)] = jnp.zeros((16,), jnp.int32)
        return 0

    lax.fori_loop(0, 16, hzero, 0)

    def key_body(v, _):
        for u in range(2):
            vv = v * 2 + u
            kb = lax.bitcast_convert_type(keysf[pl.ds(vv * 16, 16)],
                                          jnp.int32)
            m = lax.shift_right_arithmetic(kb, 31)
            ku = kb ^ (m | INT_MIN)
            valid = (_splat(vv * 16) + iota) < _splat(nq)
            ku = jnp.where(valid, ku, KEY_PAD)
            keyu[pl.ds(vv * 16, 16)] = ku
            d = lax.shift_right_logical(ku, 24)
            d_s, _ = _sorted_digits(d)
            is_end, _, cnt = _runs(d_s)
            plsc.addupdate_scatter(hist, [d_s], cnt, mask=is_end)
        return 0

    lax.fori_loop(0, NV // 2, key_body, 0)

    pltpu.sync_copy(hist, sp_hists.at[bb, q])
    plsc.subcore_barrier()
    for j in range(4):
        pltpu.sync_copy(sp_hists.at[bb, j], hists4.at[j])

    def bstar_body(v, carry):
        bstar, run = carry
        h = (hists4[0, pl.ds(v * 16, 16)] + hists4[1, pl.ds(v * 16, 16)]
             + hists4[2, pl.ds(v * 16, 16)] + hists4[3, pl.ds(v * 16, 16)])
        inc = plsc.cumsum(h) + _splat(run)
        bstar = bstar + jnp.sum((inc < K).astype(jnp.int32))
        return bstar, jnp.max(inc)

    bstar, _ = lax.fori_loop(0, 16, bstar_body,
                             (jnp.int32(0), jnp.int32(0)))
    bstar_v = _splat(bstar)

    def c1_body(v, acc):
        h = (hists4[0, pl.ds(v * 16, 16)] + hists4[1, pl.ds(v * 16, 16)]
             + hists4[2, pl.ds(v * 16, 16)] + hists4[3, pl.ds(v * 16, 16)])
        binid = _splat(v * 16) + iota
        return acc + jnp.sum(jnp.where(binid < bstar_v, h, 0))

    c1 = lax.fori_loop(0, 16, c1_body, jnp.int32(0))

    def bzero(v, _):
        bufb[pl.ds(v * 16, 16)] = jnp.full((16,), 256, jnp.int32)
        return 0

    lax.fori_loop(0, BUFB // 16, bzero, 0)

    def bcompact_body(v, off):
        ku = keyu[pl.ds(v * 16, 16)]
        d1 = lax.shift_right_logical(ku, 24)
        d2 = lax.shift_right_logical(ku, 16) & 0xFF
        match = d1 == bstar_v

        @pl.when(off <= BUFB - 16)
        def _():
            plsc.store_compressed(bufb.at[pl.ds(off, 16)], d2, mask=match)

        return off + jnp.max(plsc.all_reduce_population_count(match))

    cntb = lax.fori_loop(0, NV, bcompact_body, jnp.int32(0))

    def h2zero(v, _):
        hist[pl.ds(v * 16, 16)] = jnp.zeros((16,), jnp.int32)
        return 0

    lax.fori_loop(0, 16, h2zero, 0)

    def lvl2_body(v, _):
        for u in range(2):
            d2m = bufb[pl.ds((v * 2 + u) * 16, 16)]
            d_s, _ = _sorted_digits(d2m)
            is_end, _, cnt = _runs(d_s)
            ok = is_end & (d_s < 256)
            plsc.addupdate_scatter(hist, [jnp.minimum(d_s, 255)], cnt,
                                   mask=ok)
        return 0

    lax.fori_loop(0, (cntb + 31) // 32, lvl2_body, 0)

    plsc.subcore_barrier()
    pltpu.sync_copy(hist, sp_hists.at[bb, q])
    plsc.subcore_barrier()
    for j in range(4):
        pltpu.sync_copy(sp_hists.at[bb, j], hists4.at[j])

    k2 = K - c1

    def b2_body(v, carry):
        b2, run = carry
        h = (hists4[0, pl.ds(v * 16, 16)] + hists4[1, pl.ds(v * 16, 16)]
             + hists4[2, pl.ds(v * 16, 16)] + hists4[3, pl.ds(v * 16, 16)])
        inc = plsc.cumsum(h) + _splat(run)
        b2 = b2 + jnp.sum((inc < _splat(k2)).astype(jnp.int32))
        return b2, jnp.max(inc)

    b2star, _ = lax.fori_loop(0, 16, b2_body,
                              (jnp.int32(0), jnp.int32(0)))
    b2star_v = _splat(b2star)

    def czero(v, _):
        cand_k[pl.ds(v * 16, 16)] = jnp.full((16,), KEY_PAD, jnp.int32)
        cand_i[pl.ds(v * 16, 16)] = jnp.full((16,), IDX_PAD, jnp.int32)
        return 0

    lax.fori_loop(0, CAND // 16, czero, 0)

    def compact_body(v, off):
        ku = keyu[pl.ds(v * 16, 16)]
        d = lax.shift_right_logical(ku, 24)
        d2 = lax.shift_right_logical(ku, 16) & 0xFF
        keep = (d < bstar_v) | ((d == bstar_v) & (d2 <= b2star_v))
        node = _splat(q * SH + v * 16) + iota

        @pl.when(off <= CAND - 16)
        def _():
            plsc.store_compressed(cand_k.at[pl.ds(off, 16)], ku, mask=keep)
            plsc.store_compressed(cand_i.at[pl.ds(off, 16)], node, mask=keep)

        npop = jnp.max(plsc.all_reduce_population_count(keep))
        return off + npop

    lax.fori_loop(0, NV, compact_body, jnp.int32(0))

    pltpu.sync_copy(cand_k, sp_merge_k.at[bb, pl.ds(q * CAND, CAND)])
    pltpu.sync_copy(cand_i, sp_merge_i.at[bb, pl.ds(q * CAND, CAND)])
    plsc.subcore_barrier()

    @pl.when(q == 0)
    def _():
        pltpu.sync_copy(sp_merge_k.at[bb], sk0)
        pltpu.sync_copy(sp_merge_i.at[bb], si0)
        _radix_pass(sk0, si0, sk1, si1, bins, 0, jnp.int32(0x7FF))
        _radix_pass(sk1, si1, sk0, si0, bins, 11, jnp.int32(0x7FF))
        _radix_pass(sk0, si0, sk1, si1, bins, 22, jnp.int32(0x3FF))
        pltpu.sync_copy(si1.at[pl.ds(0, K)], sel_hbm.at[pl.ds(b * K, K)])


@functools.partial(
    pl.kernel,
    out_type=jax.ShapeDtypeStruct((B * K, D), jnp.float32),
    mesh=_mesh,
    scratch_types=[
        pltpu.VMEM((K // 4,), jnp.int32),
        pltpu.VMEM((4, 128), jnp.int32),
        pltpu.VMEM((512, D), jnp.float32),
        pltpu.SemaphoreType.DMA,
    ],
    compiler_params=_params,
)
def _gather_kernel(z_hbm, sel_hbm, out_hbm, myidx, rowidx, rows, sem):
    c = lax.axis_index("c")
    s = lax.axis_index("s")
    wid = c * 16 + s
    b = wid // 4
    kq = K // 4

    pltpu.sync_copy(sel_hbm.at[pl.ds(wid * kq, kq)], myidx)
    base_row = b * N
    for v in range(kq // 16):
        nd = myidx[pl.ds(v * 16, 16)] + _splat(base_row)
        rowidx[v // 8, pl.ds((v % 8) * 16, 16)] = nd

    for ch in range(4):
        pltpu.async_copy(z_hbm.at[rowidx.at[ch]],
                         rows.at[pl.ds(ch * 128, 128)], sem)
    pltpu.make_async_copy(z_hbm.at[pl.ds(0, 512)], rows, sem).wait()
    pltpu.sync_copy(rows, out_hbm.at[pl.ds(wid * kq, kq)])


def kernel(Z):
    sel = _select_kernel(Z.reshape(-1))
    out = _gather_kernel(Z.reshape(B * N, D), sel)
    return out.reshape(B, K, D)


# speedup vs baseline: 7.5759x; 1.2968x over previous
import functools

import jax
import jax.numpy as jnp
import numpy as np
from jax import lax
from jax.experimental import pallas as pl
from jax.experimental.pallas import tpu as pltpu
from jax.experimental.pallas import tpu_sc as plsc

B = 8
N = 50000
D = 128
K = 2048

SH = 12504
SH_LAST = N - 3 * SH
NV = (SH + 15) // 16
NQP = NV * 16
NCH = (NQP + 127) // 128
KBUF = NCH * 128
CAND = 1024
M = 4 * CAND
BINS = 2048
INT_MIN = np.int32(-2147483648)
KEY_PAD = np.int32(-1)
IDX_PAD = np.int32(0x7FFFFFFF)

_mesh = plsc.VectorSubcoreMesh(core_axis_name="c", subcore_axis_name="s")
_params = pltpu.CompilerParams(needs_layout_passes=False)


def _iota():
    return lax.iota(jnp.int32, 16)


def _vtake(x, i):
    return lax.gather(
        x, i[:, None],
        dimension_numbers=lax.GatherDimensionNumbers(
            offset_dims=(), collapsed_slice_dims=(0,), start_index_map=(0,)),
        slice_sizes=(1,),
        mode=lax.GatherScatterMode.PROMISE_IN_BOUNDS)


def _splat(s):
    return jnp.full((16,), s, jnp.int32)


def _runs(d_s):
    iota = _iota()
    nxt = _vtake(d_s, jnp.minimum(iota + 1, 15))
    prv = _vtake(d_s, jnp.maximum(iota - 1, 0))
    is_end = (iota == 15) | (d_s != nxt)
    is_start = (iota == 0) | (d_s != prv)
    start = plsc.cummax(jnp.where(is_start, iota, 0))
    run_rank = iota - start
    return is_end, run_rank, run_rank + 1


def _sorted_digits(d):
    iota = _iota()
    sk, _ = plsc.sort_key_val(d * 16 + iota, d)
    return lax.shift_right_logical(sk, 4), sk & 15


def _radix_pass(src_k, src_i, dst_k, dst_i, bins, shift, mask):
    nv = M // 16

    def zero_body(v, _):
        bins[pl.ds(v * 16, 16)] = jnp.zeros((16,), jnp.int32)
        return 0

    lax.fori_loop(0, BINS // 16, zero_body, 0)

    ones = _splat(1)

    def hist_body(v, _):
        for u in range(2):
            ku = src_k[pl.ds((v * 2 + u) * 16, 16)]
            d = lax.shift_right_logical(ku, shift) & mask
            plsc.addupdate_scatter(bins, [d], ones)
        return 0

    lax.fori_loop(0, nv // 2, hist_body, 0)

    def scan_body(v, carry):
        h = bins[pl.ds(v * 16, 16)]
        inc = plsc.cumsum(h)
        bins[pl.ds(v * 16, 16)] = inc - h + _splat(carry)
        return carry + jnp.max(inc)

    lax.fori_loop(0, BINS // 16, scan_body, jnp.int32(0))

    def perm_body(v, _):
        ku = src_k[pl.ds(v * 16, 16)]
        ix = src_i[pl.ds(v * 16, 16)]
        d = lax.shift_right_logical(ku, shift) & mask
        d_s, ol = _sorted_digits(d)
        is_end, run_rank, cnt = _runs(d_s)
        base = plsc.load_gather(bins, [d_s])
        pos = base + run_rank
        plsc.store_scatter(dst_k, [pos], _vtake(ku, ol))
        plsc.store_scatter(dst_i, [pos], _vtake(ix, ol))
        plsc.addupdate_scatter(bins, [d_s], cnt, mask=is_end)
        return 0

    lax.fori_loop(0, nv, perm_body, 0)


@functools.partial(
    pl.kernel,
    out_type=jax.ShapeDtypeStruct((B * K,), jnp.int32),
    mesh=_mesh,
    scratch_types=[
        pltpu.VMEM((KBUF,), jnp.float32),
        pltpu.VMEM((KBUF,), jnp.int32),
        pltpu.VMEM((NQP,), jnp.int32),
        pltpu.VMEM((256,), jnp.int32),
        pltpu.VMEM((4, 256), jnp.int32),
        pltpu.VMEM((CAND,), jnp.int32),
        pltpu.VMEM((CAND,), jnp.int32),
        pltpu.VMEM((M,), jnp.int32),
        pltpu.VMEM((M,), jnp.int32),
        pltpu.VMEM((M,), jnp.int32),
        pltpu.VMEM((M,), jnp.int32),
        pltpu.VMEM((BINS,), jnp.int32),
        pltpu.VMEM_SHARED((4, 4, 256), jnp.int32),
        pltpu.VMEM_SHARED((4, M), jnp.int32),
        pltpu.VMEM_SHARED((4, M), jnp.int32),
        pltpu.SemaphoreType.DMA,
    ],
    compiler_params=_params,
)
def _select_kernel(z_flat, sel_hbm, keysf, gidx, keyu, hist, hists4,
                   cand_k, cand_i, sk0, si0, sk1, si1, bins,
                   sp_hists, sp_merge_k, sp_merge_i, sem):
    c = lax.axis_index("c")
    s = lax.axis_index("s")
    wid = c * 16 + s
    b = wid // 4
    bb = s // 4
    q = s % 4
    iota = _iota()

    row0 = b * N + q * SH
    nq = jnp.where(q == 3, SH_LAST, SH)
    nclamp = _splat(nq - 1)

    def fire_body(ch, _):
        for j in range(8):
            i = _splat(ch * 128 + j * 16) + iota
            r = _splat(row0) + jnp.minimum(i, nclamp)
            gidx[pl.ds(ch * 128 + j * 16, 16)] = r * D + (D - 1)
        pltpu.async_copy(z_flat.at[gidx.at[pl.ds(ch * 128, 128)]],
                         keysf.at[pl.ds(ch * 128, 128)], sem)
        return 0

    lax.fori_loop(0, NCH, fire_body, 0)
    pltpu.make_async_copy(z_flat.at[pl.ds(0, KBUF)], keysf, sem).wait()

    def hzero(v, _):
        hist[pl.ds(v * 16, 16)] = jnp.zeros((16,), jnp.int32)
        return 0

    lax.fori_loop(0, 16, hzero, 0)

    ones = _splat(1)

    def key_body(v, _):
        for u in range(2):
            vv = v * 2 + u
            kb = lax.bitcast_convert_type(keysf[pl.ds(vv * 16, 16)],
                                          jnp.int32)
            m = lax.shift_right_arithmetic(kb, 31)
            ku = kb ^ (m | INT_MIN)
            valid = (_splat(vv * 16) + iota) < _splat(nq)
            ku = jnp.where(valid, ku, KEY_PAD)
            keyu[pl.ds(vv * 16, 16)] = ku
            d = lax.shift_right_logical(ku, 24)
            plsc.addupdate_scatter(hist, [d], ones)
        return 0

    lax.fori_loop(0, NV // 2, key_body, 0)

    pltpu.sync_copy(hist, sp_hists.at[bb, q])
    plsc.subcore_barrier()
    for j in range(4):
        pltpu.sync_copy(sp_hists.at[bb, j], hists4.at[j])

    def bstar_body(v, carry):
        bstar, run = carry
        h = (hists4[0, pl.ds(v * 16, 16)] + hists4[1, pl.ds(v * 16, 16)]
             + hists4[2, pl.ds(v * 16, 16)] + hists4[3, pl.ds(v * 16, 16)])
        inc = plsc.cumsum(h) + _splat(run)
        bstar = bstar + jnp.sum((inc < K).astype(jnp.int32))
        return bstar, jnp.max(inc)

    bstar, _ = lax.fori_loop(0, 16, bstar_body,
                             (jnp.int32(0), jnp.int32(0)))
    bstar_v = _splat(bstar)

    def c1_body(v, acc):
        h = (hists4[0, pl.ds(v * 16, 16)] + hists4[1, pl.ds(v * 16, 16)]
             + hists4[2, pl.ds(v * 16, 16)] + hists4[3, pl.ds(v * 16, 16)])
        binid = _splat(v * 16) + iota
        return acc + jnp.sum(jnp.where(binid < bstar_v, h, 0))

    c1 = lax.fori_loop(0, 16, c1_body, jnp.int32(0))

    def h2zero(v, _):
        hist[pl.ds(v * 16, 16)] = jnp.zeros((16,), jnp.int32)
        return 0

    lax.fori_loop(0, 16, h2zero, 0)

    def lvl2_body(v, _):
        for u in range(2):
            ku = keyu[pl.ds((v * 2 + u) * 16, 16)]
            d1 = lax.shift_right_logical(ku, 24)
            d2 = lax.shift_right_logical(ku, 16) & 0xFF
            plsc.addupdate_scatter(hist, [d2], ones, mask=d1 == bstar_v)
        return 0

    lax.fori_loop(0, NV // 2, lvl2_body, 0)

    plsc.subcore_barrier()
    pltpu.sync_copy(hist, sp_hists.at[bb, q])
    plsc.subcore_barrier()
    for j in range(4):
        pltpu.sync_copy(sp_hists.at[bb, j], hists4.at[j])

    k2 = K - c1

    def b2_body(v, carry):
        b2, run = carry
        h = (hists4[0, pl.ds(v * 16, 16)] + hists4[1, pl.ds(v * 16, 16)]
             + hists4[2, pl.ds(v * 16, 16)] + hists4[3, pl.ds(v * 16, 16)])
        inc = plsc.cumsum(h) + _splat(run)
        b2 = b2 + jnp.sum((inc < _splat(k2)).astype(jnp.int32))
        return b2, jnp.max(inc)

    b2star, _ = lax.fori_loop(0, 16, b2_body,
                              (jnp.int32(0), jnp.int32(0)))
    b2star_v = _splat(b2star)

    def czero(v, _):
        cand_k[pl.ds(v * 16, 16)] = jnp.full((16,), KEY_PAD, jnp.int32)
        cand_i[pl.ds(v * 16, 16)] = jnp.full((16,), IDX_PAD, jnp.int32)
        return 0

    lax.fori_loop(0, CAND // 16, czero, 0)

    def compact_body(v, off):
        ku = keyu[pl.ds(v * 16, 16)]
        d = lax.shift_right_logical(ku, 24)
        d2 = lax.shift_right_logical(ku, 16) & 0xFF
        keep = (d < bstar_v) | ((d == bstar_v) & (d2 <= b2star_v))
        node = _splat(q * SH + v * 16) + iota

        @pl.when(off <= CAND - 16)
        def _():
            plsc.store_compressed(cand_k.at[pl.ds(off, 16)], ku, mask=keep)
            plsc.store_compressed(cand_i.at[pl.ds(off, 16)], node, mask=keep)

        npop = jnp.max(plsc.all_reduce_population_count(keep))
        return off + npop

    lax.fori_loop(0, NV, compact_body, jnp.int32(0))

    pltpu.sync_copy(cand_k, sp_merge_k.at[bb, pl.ds(q * CAND, CAND)])
    pltpu.sync_copy(cand_i, sp_merge_i.at[bb, pl.ds(q * CAND, CAND)])
    plsc.subcore_barrier()

    @pl.when(q == 0)
    def _():
        pltpu.sync_copy(sp_merge_k.at[bb], sk0)
        pltpu.sync_copy(sp_merge_i.at[bb], si0)
        _radix_pass(sk0, si0, sk1, si1, bins, 0, jnp.int32(0x7FF))
        _radix_pass(sk1, si1, sk0, si0, bins, 11, jnp.int32(0x7FF))
        _radix_pass(sk0, si0, sk1, si1, bins, 22, jnp.int32(0x3FF))
        pltpu.sync_copy(si1.at[pl.ds(0, K)], sel_hbm.at[pl.ds(b * K, K)])


@functools.partial(
    pl.kernel,
    out_type=jax.ShapeDtypeStruct((B * K, D), jnp.float32),
    mesh=_mesh,
    scratch_types=[
        pltpu.VMEM((K // 4,), jnp.int32),
        pltpu.VMEM((4, 128), jnp.int32),
        pltpu.VMEM((512, D), jnp.float32),
        pltpu.SemaphoreType.DMA,
    ],
    compiler_params=_params,
)
def _gather_kernel(z_hbm, sel_hbm, out_hbm, myidx, rowidx, rows, sem):
    c = lax.axis_index("c")
    s = lax.axis_index("s")
    wid = c * 16 + s
    b = wid // 4
    kq = K // 4

    pltpu.sync_copy(sel_hbm.at[pl.ds(wid * kq, kq)], myidx)
    base_row = b * N
    for v in range(kq // 16):
        nd = myidx[pl.ds(v * 16, 16)] + _splat(base_row)
        rowidx[v // 8, pl.ds((v % 8) * 16, 16)] = nd

    for ch in range(4):
        pltpu.async_copy(z_hbm.at[rowidx.at[ch]],
                         rows.at[pl.ds(ch * 128, 128)], sem)
    pltpu.make_async_copy(z_hbm.at[pl.ds(0, 512)], rows, sem).wait()
    pltpu.sync_copy(rows, out_hbm.at[pl.ds(wid * kq, kq)])


def kernel(Z):
    sel = _select_kernel(Z.reshape(-1))
    out = _gather_kernel(Z.reshape(B * N, D), sel)
    return out.reshape(B, K, D)
